# baseline (speedup 1.0000x reference)
import functools

import jax
import jax.numpy as jnp
from jax import lax
from jax.experimental import pallas as pl
from jax.experimental.pallas import tpu as pltpu
from jax.experimental.pallas import tpu_sc as plsc

B = 4096
L = 200
EMB = 64
HID = 128

NC = 2
NS = 16
NW = NC * NS
BPW = B // NW
GATHER = 100
STEPS = BPW * L // GATHER
NBUF = 4
ROWS_PER_ACC = 10


def _cbow_pool_build():
    mesh = plsc.VectorSubcoreMesh(core_axis_name="c", subcore_axis_name="s")

    @functools.partial(
        pl.kernel,
        out_type=jax.ShapeDtypeStruct((B, EMB), jnp.float32),
        mesh=mesh,
        compiler_params=pltpu.CompilerParams(use_tc_tiling_on_sc=False),
        scratch_types=[
            pltpu.VMEM((STEPS, GATHER), jnp.int32),
            pltpu.VMEM((NBUF, GATHER, EMB), jnp.float32),
            pltpu.VMEM((BPW, EMB), jnp.float32),
            pltpu.SemaphoreType.DMA,
            pltpu.SemaphoreType.DMA,
            pltpu.SemaphoreType.DMA,
            pltpu.SemaphoreType.DMA,
        ],
    )
    def pool(words_hbm, emb_hbm, out_hbm, idx_v, rows_v, hout_v, s0, s1, s2, s3):
        sems = [s0, s1, s2, s3]
        wid = lax.axis_index("s") * NC + lax.axis_index("c")

        pltpu.sync_copy(words_hbm.at[pl.ds(wid * STEPS, STEPS)], idx_v)

        for s in range(NBUF):
            pltpu.async_copy(emb_hbm.at[idx_v.at[s]], rows_v.at[s], sems[s])

        def accum(s, init):
            def body(r10, acc):
                a0, a1, a2, a3 = acc
                for u in range(ROWS_PER_ACC):
                    r = r10 * ROWS_PER_ACC + u
                    a0 = a0 + rows_v[s, r, pl.ds(0, 16)]
                    a1 = a1 + rows_v[s, r, pl.ds(16, 16)]
                    a2 = a2 + rows_v[s, r, pl.ds(32, 16)]
                    a3 = a3 + rows_v[s, r, pl.ds(48, 16)]
                return (a0, a1, a2, a3)
            return lax.fori_loop(0, GATHER // ROWS_PER_ACC, body, init)

        zeros4 = tuple(jnp.zeros((16,), jnp.float32) for _ in range(4))

        def wait(s):
            pltpu.make_async_copy(
                emb_hbm.at[idx_v.at[0]], rows_v.at[s], sems[s]
            ).wait()

        def step_block(i, issue_next):
            stash = zeros4
            for s in range(NBUF):
                wait(s)
                if s % 2 == 0:
                    stash = accum(s, zeros4)
                else:
                    acc = accum(s, stash)
                    b = i * 2 + (s // 2)
                    hout_v[b, pl.ds(0, 16)] = acc[0]
                    hout_v[b, pl.ds(16, 16)] = acc[1]
                    hout_v[b, pl.ds(32, 16)] = acc[2]
                    hout_v[b, pl.ds(48, 16)] = acc[3]
                if issue_next:
                    jn = i * NBUF + s + NBUF
                    pltpu.async_copy(emb_hbm.at[idx_v.at[jn]], rows_v.at[s], sems[s])

        def main_body(i, carry):
            step_block(i, True)
            return carry

        lax.fori_loop(0, STEPS // NBUF - 1, main_body, 0)
        step_block(STEPS // NBUF - 1, False)

        pltpu.sync_copy(hout_v, out_hbm.at[pl.ds(wid * BPW, BPW)])

    return pool


_cbow_pool = _cbow_pool_build()


def _mlp_body(h_ref, w0_ref, b0_ref, w1_ref, b1_ref, wout_ref, bout_ref, o_ref):
    h = h_ref[...]
    t = jnp.tanh(jnp.dot(h, w0_ref[...], preferred_element_type=jnp.float32)
                 + b0_ref[...])
    t = jnp.tanh(jnp.dot(t, w1_ref[...], preferred_element_type=jnp.float32)
                 + b1_ref[...])
    o_ref[...] = (jnp.sum(t * wout_ref[...], axis=1, keepdims=True)
                  + bout_ref[...])


def kernel(words, emb, W0, b0, W1, b1, Wout, bout):
    words32 = words.astype(jnp.int32).reshape(B * L // GATHER, GATHER)
    h = _cbow_pool(words32, emb)
    out = pl.pallas_call(
        _mlp_body,
        out_shape=jax.ShapeDtypeStruct((B, 1), jnp.float32),
    )(
        h,
        W0,
        b0.reshape(1, HID),
        W1,
        b1.reshape(1, HID),
        Wout.reshape(1, HID),
        bout.reshape(1, 1),
    )
    return out


# speedup vs baseline: 1.0243x; 1.0243x over previous
import functools

import jax
import jax.numpy as jnp
from jax import lax
from jax.experimental import pallas as pl
from jax.experimental.pallas import tpu as pltpu
from jax.experimental.pallas import tpu_sc as plsc

B = 4096
L = 200
EMB = 64
HID = 128

NC = 2
NS = 16
NW = NC * NS
BPW = B // NW
GATHER = L
STEPS = BPW
NBUF = 4
ROWS_PER_ACC = 10


def _cbow_pool_build():
    mesh = plsc.VectorSubcoreMesh(core_axis_name="c", subcore_axis_name="s")

    @functools.partial(
        pl.kernel,
        out_type=jax.ShapeDtypeStruct((B, EMB), jnp.float32),
        mesh=mesh,
        compiler_params=pltpu.CompilerParams(use_tc_tiling_on_sc=False),
        scratch_types=[
            pltpu.VMEM((BPW, L), jnp.int32),
            pltpu.VMEM((NBUF, GATHER, EMB), jnp.float32),
            pltpu.VMEM((BPW, EMB), jnp.float32),
            pltpu.SemaphoreType.DMA,
            pltpu.SemaphoreType.DMA,
            pltpu.SemaphoreType.DMA,
            pltpu.SemaphoreType.DMA,
        ],
    )
    def pool(words_hbm, emb_hbm, out_hbm, idx_v, rows_v, hout_v, s0, s1, s2, s3):
        sems = [s0, s1, s2, s3]
        wid = lax.axis_index("s") * NC + lax.axis_index("c")

        pltpu.sync_copy(words_hbm.at[pl.ds(wid * BPW, BPW)], idx_v)

        for s in range(NBUF):
            pltpu.async_copy(emb_hbm.at[idx_v.at[s]], rows_v.at[s], sems[s])

        def accum(s):
            def body(r10, acc):
                a0, a1, a2, a3 = acc
                for u in range(ROWS_PER_ACC):
                    r = r10 * ROWS_PER_ACC + u
                    a0 = a0 + rows_v[s, r, pl.ds(0, 16)]
                    a1 = a1 + rows_v[s, r, pl.ds(16, 16)]
                    a2 = a2 + rows_v[s, r, pl.ds(32, 16)]
                    a3 = a3 + rows_v[s, r, pl.ds(48, 16)]
                return (a0, a1, a2, a3)
            zeros4 = tuple(jnp.zeros((16,), jnp.float32) for _ in range(4))
            return lax.fori_loop(0, GATHER // ROWS_PER_ACC, body, zeros4)

        def wait(s):
            pltpu.make_async_copy(
                emb_hbm.at[idx_v.at[0]], rows_v.at[s], sems[s]
            ).wait()

        def step_block(i, issue_next):
            for s in range(NBUF):
                wait(s)
                acc = accum(s)
                b = i * NBUF + s
                hout_v[b, pl.ds(0, 16)] = acc[0]
                hout_v[b, pl.ds(16, 16)] = acc[1]
                hout_v[b, pl.ds(32, 16)] = acc[2]
                hout_v[b, pl.ds(48, 16)] = acc[3]
                if issue_next:
                    pltpu.async_copy(
                        emb_hbm.at[idx_v.at[b + NBUF]], rows_v.at[s], sems[s]
                    )

        def main_body(i, carry):
            step_block(i, True)
            return carry

        lax.fori_loop(0, STEPS // NBUF - 1, main_body, 0)
        step_block(STEPS // NBUF - 1, False)

        pltpu.sync_copy(hout_v, out_hbm.at[pl.ds(wid * BPW, BPW)])

    return pool


_cbow_pool = _cbow_pool_build()


def _mlp_body(h_ref, w0_ref, b0_ref, w1_ref, b1_ref, wout_ref, bout_ref, o_ref):
    h = h_ref[...]
    t = jnp.tanh(jnp.dot(h, w0_ref[...], preferred_element_type=jnp.float32)
                 + b0_ref[...])
    t = jnp.tanh(jnp.dot(t, w1_ref[...], preferred_element_type=jnp.float32)
                 + b1_ref[...])
    o_ref[...] = (jnp.sum(t * wout_ref[...], axis=1, keepdims=True)
                  + bout_ref[...])


def kernel(words, emb, W0, b0, W1, b1, Wout, bout):
    h = _cbow_pool(words.astype(jnp.int32), emb)
    out = pl.pallas_call(
        _mlp_body,
        out_shape=jax.ShapeDtypeStruct((B, 1), jnp.float32),
    )(
        h,
        W0,
        b0.reshape(1, HID),
        W1,
        b1.reshape(1, HID),
        Wout.reshape(1, HID),
        bout.reshape(1, 1),
    )
    return out


# speedup vs baseline: 1.0246x; 1.0003x over previous
import functools

import jax
import jax.numpy as jnp
from jax import lax
from jax.experimental import pallas as pl
from jax.experimental.pallas import tpu as pltpu
from jax.experimental.pallas import tpu_sc as plsc

B = 4096
L = 200
EMB = 64
HID = 128

NC = 2
NS = 16
NW = NC * NS
BPW = B // NW
GATHER = L
STEPS = BPW
NBUF = 4
ROWS_PER_ACC = 10


def _cbow_pool_build():
    mesh = plsc.VectorSubcoreMesh(core_axis_name="c", subcore_axis_name="s")

    @functools.partial(
        pl.kernel,
        out_type=jax.ShapeDtypeStruct((B, EMB), jnp.float32),
        mesh=mesh,
        compiler_params=pltpu.CompilerParams(use_tc_tiling_on_sc=False),
        scratch_types=[
            pltpu.VMEM((BPW, 128), jnp.int32),
            pltpu.VMEM((BPW, 128), jnp.int32),
            pltpu.VMEM((NBUF, GATHER, EMB), jnp.float32),
            pltpu.VMEM((BPW, EMB), jnp.float32),
            pltpu.SemaphoreType.DMA,
            pltpu.SemaphoreType.DMA,
            pltpu.SemaphoreType.DMA,
            pltpu.SemaphoreType.DMA,
        ],
    )
    def pool(wa_hbm, wb_hbm, emb_hbm, out_hbm, ia_v, ib_v, rows_v, hout_v,
             s0, s1, s2, s3):
        sems = [s0, s1, s2, s3]
        wid = lax.axis_index("s") * NC + lax.axis_index("c")

        pltpu.sync_copy(wa_hbm.at[pl.ds(wid * BPW, BPW)], ia_v)
        pltpu.sync_copy(wb_hbm.at[pl.ds(wid * BPW, BPW)], ib_v)

        def issue(b, s):
            pltpu.async_copy(
                emb_hbm.at[ia_v.at[b]], rows_v.at[s, pl.ds(0, 128)], sems[s]
            )
            pltpu.async_copy(
                emb_hbm.at[ib_v.at[b, pl.ds(56, 72)]],
                rows_v.at[s, pl.ds(128, 72)],
                sems[s],
            )

        for s in range(NBUF):
            issue(s, s)

        def accum(s):
            def body(r10, acc):
                a0, a1, a2, a3 = acc
                for u in range(ROWS_PER_ACC):
                    r = r10 * ROWS_PER_ACC + u
                    a0 = a0 + rows_v[s, r, pl.ds(0, 16)]
                    a1 = a1 + rows_v[s, r, pl.ds(16, 16)]
                    a2 = a2 + rows_v[s, r, pl.ds(32, 16)]
                    a3 = a3 + rows_v[s, r, pl.ds(48, 16)]
                return (a0, a1, a2, a3)
            zeros4 = tuple(jnp.zeros((16,), jnp.float32) for _ in range(4))
            return lax.fori_loop(0, GATHER // ROWS_PER_ACC, body, zeros4)

        def wait(s):
            pltpu.make_async_copy(
                emb_hbm.at[ia_v.at[0]], rows_v.at[s, pl.ds(0, 128)], sems[s]
            ).wait()
            pltpu.make_async_copy(
                emb_hbm.at[ib_v.at[0, pl.ds(56, 72)]],
                rows_v.at[s, pl.ds(128, 72)],
                sems[s],
            ).wait()

        def step_block(i, issue_next):
            for s in range(NBUF):
                wait(s)
                acc = accum(s)
                b = i * NBUF + s
                hout_v[b, pl.ds(0, 16)] = acc[0]
                hout_v[b, pl.ds(16, 16)] = acc[1]
                hout_v[b, pl.ds(32, 16)] = acc[2]
                hout_v[b, pl.ds(48, 16)] = acc[3]
                if issue_next:
                    issue(b + NBUF, s)

        def main_body(i, carry):
            step_block(i, True)
            return carry

        lax.fori_loop(0, STEPS // NBUF - 1, main_body, 0)
        step_block(STEPS // NBUF - 1, False)

        pltpu.sync_copy(hout_v, out_hbm.at[pl.ds(wid * BPW, BPW)])

    return pool


_cbow_pool = _cbow_pool_build()


def _mlp_body(h_ref, w0_ref, b0_ref, w1_ref, b1_ref, wout_ref, bout_ref, o_ref):
    h = h_ref[...]
    t = jnp.tanh(jnp.dot(h, w0_ref[...], preferred_element_type=jnp.float32)
                 + b0_ref[...])
    t = jnp.tanh(jnp.dot(t, w1_ref[...], preferred_element_type=jnp.float32)
                 + b1_ref[...])
    o_ref[...] = (jnp.sum(t * wout_ref[...], axis=1, keepdims=True)
                  + bout_ref[...])


def kernel(words, emb, W0, b0, W1, b1, Wout, bout):
    w32 = words.astype(jnp.int32)
    wa = w32[:, :128]
    wb = w32[:, L - 128:]
    h = _cbow_pool(wa, wb, emb)
    out = pl.pallas_call(
        _mlp_body,
        out_shape=jax.ShapeDtypeStruct((B, 1), jnp.float32),
    )(
        h,
        W0,
        b0.reshape(1, HID),
        W1,
        b1.reshape(1, HID),
        Wout.reshape(1, HID),
        bout.reshape(1, 1),
    )
    return out
